# baseline (speedup 1.0000x reference)
import functools

import jax
import jax.numpy as jnp
from jax import lax
from jax.experimental import pallas as pl
from jax.experimental.pallas import tpu as pltpu
from jax.experimental.pallas import tpu_sc as plsc

B = 16384
D = 128
NTOK = 10
NB = 10

_NC = 2
_NS = 16
_NW = _NC * _NS
_CPW = B // _NW
_CH = 64
_NCHUNK = _CPW // _CH
_GROWS = _CH * NTOK
_NGD = _GROWS // 128

_OFF_CLS = 0
_OFF_WM = 1
_OFF_WP = 2
_OFF_SAT1 = 3
_OFF_SAT2 = 6
_OFF_OBJ1 = 9
_OFF_OBJ2 = 12
_OFF_ORG1 = 15
_OFF_ORG2 = 25
_OFF_BB = 35
_TROWS = 36


def _splat(val):
    return jnp.full((16,), val, dtype=jnp.int32)


def _body(T_hbm, s1_hbm, s2_hbm, o1_hbm, o2_hbm, g1_hbm, g2_hbm,
          miss_hbm, pc_hbm, bool_hbm, Wb_hbm, bm_hbm, bp_hbm,
          out_hbm,
          outb, gidx, idxb, scal, boolb, Wb_v, bm_v, bp_v, sem):
    wid = lax.axis_index("s") * _NC + lax.axis_index("c")

    pltpu.sync_copy(Wb_hbm, Wb_v)
    pltpu.sync_copy(bm_hbm, bm_v)
    pltpu.sync_copy(bp_hbm, bp_v)

    def chunk_body(c, carry):
        base = wid * _CPW + c * _CH

        pltpu.sync_copy(s1_hbm.at[pl.ds(base, _CH)], idxb.at[0])
        pltpu.sync_copy(s2_hbm.at[pl.ds(base, _CH)], idxb.at[1])
        pltpu.sync_copy(o1_hbm.at[pl.ds(base, _CH)], idxb.at[2])
        pltpu.sync_copy(o2_hbm.at[pl.ds(base, _CH)], idxb.at[3])
        pltpu.sync_copy(g1_hbm.at[pl.ds(base, _CH)], idxb.at[4])
        pltpu.sync_copy(g2_hbm.at[pl.ds(base, _CH)], idxb.at[5])
        pltpu.sync_copy(miss_hbm.at[pl.ds(base, _CH)], scal.at[0])
        pltpu.sync_copy(pc_hbm.at[pl.ds(base, _CH)], scal.at[1])
        pltpu.sync_copy(bool_hbm.at[pl.ds(base, _CH), :], boolb)

        for g in range(_CH // 16):
            rows = jnp.full((16,), g * 16, jnp.int32) + lax.iota(jnp.int32, 16)
            pos0 = rows * NTOK

            def put(t, val):
                p = pos0 + t
                plsc.store_scatter(gidx, [p >> 7, p & 127], val)

            put(0, _splat(_OFF_CLS))
            put(1, _splat(_OFF_WM))
            put(2, _splat(_OFF_WP))
            put(9, _splat(_OFF_BB))
            v = plsc.load_gather(idxb, [_splat(0), rows])
            put(3, v + _OFF_SAT1)
            v = plsc.load_gather(idxb, [_splat(1), rows])
            put(4, v + _OFF_SAT2)
            v = plsc.load_gather(idxb, [_splat(2), rows])
            put(5, v + _OFF_OBJ1)
            v = plsc.load_gather(idxb, [_splat(3), rows])
            put(6, v + _OFF_OBJ2)
            v = plsc.load_gather(idxb, [_splat(4), rows])
            put(7, v + _OFF_ORG1)
            v = plsc.load_gather(idxb, [_splat(5), rows])
            put(8, v + _OFF_ORG2)

        copies = []
        for j in range(_NGD):
            copies.append(pltpu.async_copy(
                T_hbm.at[gidx.at[j]], outb.at[pl.ds(j * 128, 128)], sem))
        for cp in copies:
            cp.wait()

        def row_body(i, rc):
            mv = plsc.load_gather(scal, [_splat(0), jnp.full((16,), i, jnp.int32)])
            pv = plsc.load_gather(scal, [_splat(1), jnp.full((16,), i, jnp.int32)])
            bvs = [plsc.load_gather(
                boolb, [jnp.full((16,), i, jnp.int32), _splat(j)])
                for j in range(NB)]
            r1 = i * NTOK + 1
            r2 = i * NTOK + 2
            r9 = i * NTOK + 9
            for k in range(D // 16):
                sl = pl.ds(k * 16, 16)
                outb[r1, sl] = outb[r1, sl] * mv + bm_v[sl]
                outb[r2, sl] = outb[r2, sl] * pv + bp_v[sl]
                acc = outb[r9, sl]
                for j in range(NB):
                    acc = acc + bvs[j] * Wb_v[j, sl]
                outb[r9, sl] = acc
            return rc

        lax.fori_loop(0, _CH, row_body, 0)

        pltpu.sync_copy(outb, out_hbm.at[pl.ds(base * NTOK, _GROWS)])
        return carry

    lax.fori_loop(0, _NCHUNK, chunk_body, 0)


@functools.partial(jax.jit, static_argnames=())
def _run(T, s1, s2, o1, o2, g1, g2, miss, pc, bools, Wb, bm, bp):
    call = functools.partial(
        pl.kernel,
        out_type=jax.ShapeDtypeStruct((B * NTOK, D), jnp.float32),
        mesh=plsc.VectorSubcoreMesh(core_axis_name="c", subcore_axis_name="s"),
        compiler_params=pltpu.CompilerParams(needs_layout_passes=False),
        scratch_types=[
            pltpu.VMEM((_GROWS, D), jnp.float32),
            pltpu.VMEM((_NGD, 128), jnp.int32),
            pltpu.VMEM((6, _CH), jnp.int32),
            pltpu.VMEM((2, _CH), jnp.float32),
            pltpu.VMEM((_CH, NB), jnp.float32),
            pltpu.VMEM((NB, D), jnp.float32),
            pltpu.VMEM((D,), jnp.float32),
            pltpu.VMEM((D,), jnp.float32),
            pltpu.SemaphoreType.DMA,
        ],
    )(_body)
    return call(T, s1, s2, o1, o2, g1, g2, miss, pc, bools, Wb, bm, bp)


def kernel(miss_distance, pc, sat1_type, sat2_type, obj1_type, obj2_type,
           org1, org2, bool_features, W_miss, b_miss, W_pc, b_pc,
           E_sat1, E_sat2, E_obj1, E_obj2, E_org1, E_org2, W_bool, b_bool,
           CLS):
    T = jnp.concatenate([
        CLS.reshape(1, D), W_miss, W_pc,
        E_sat1, E_sat2, E_obj1, E_obj2, E_org1, E_org2,
        b_bool.reshape(1, D),
    ], axis=0)
    out = _run(
        T,
        sat1_type.astype(jnp.int32), sat2_type.astype(jnp.int32),
        obj1_type.astype(jnp.int32), obj2_type.astype(jnp.int32),
        org1.astype(jnp.int32), org2.astype(jnp.int32),
        miss_distance.reshape(B), pc.reshape(B), bool_features,
        W_bool, b_miss, b_pc)
    return out.reshape(B, NTOK, D)


# speedup vs baseline: 2.3759x; 2.3759x over previous
import functools

import jax
import jax.numpy as jnp
from jax import lax
from jax.experimental import pallas as pl
from jax.experimental.pallas import tpu as pltpu
from jax.experimental.pallas import tpu_sc as plsc

B = 16384
D = 128
NTOK = 10
NB = 10

_NC = 2
_NS = 16
_NW = _NC * _NS
_CPW = B // _NW
_CH = 8
_NPAIR = _CPW // (2 * _CH)

_OFF_CLS = 0
_OFF_WM = 1
_OFF_WP = 2
_OFF_EMB = (3, 6, 9, 12, 15, 25)
_OFF_BB = 35
_OFF_WB = 36
_OFF_BM = 46
_OFF_BP = 47
_TROWS = 48


def _body(T_hbm, s1_hbm, s2_hbm, o1_hbm, o2_hbm, g1_hbm, g2_hbm,
          miss_hbm, pc_hbm, boolT_hbm, out_hbm,
          tv, idxv, scal, boolv, ob0, ob1, sem0, sem1):
    wid = lax.axis_index("s") * _NC + lax.axis_index("c")
    base = wid * _CPW

    pltpu.sync_copy(T_hbm, tv)
    pltpu.sync_copy(s1_hbm.at[pl.ds(base, _CPW)], idxv.at[0])
    pltpu.sync_copy(s2_hbm.at[pl.ds(base, _CPW)], idxv.at[1])
    pltpu.sync_copy(o1_hbm.at[pl.ds(base, _CPW)], idxv.at[2])
    pltpu.sync_copy(o2_hbm.at[pl.ds(base, _CPW)], idxv.at[3])
    pltpu.sync_copy(g1_hbm.at[pl.ds(base, _CPW)], idxv.at[4])
    pltpu.sync_copy(g2_hbm.at[pl.ds(base, _CPW)], idxv.at[5])
    pltpu.sync_copy(miss_hbm.at[pl.ds(base, _CPW)], scal.at[0])
    pltpu.sync_copy(pc_hbm.at[pl.ds(base, _CPW)], scal.at[1])
    pltpu.sync_copy(boolT_hbm.at[:, pl.ds(base, _CPW)], boolv)

    for ob in (ob0, ob1):
        for l in range(_CH):
            for k in range(D // 16):
                ob[l, 0, pl.ds(k * 16, 16)] = tv[_OFF_CLS, pl.ds(k * 16, 16)]

    def pair_body(c2, carry):
        r0 = c2 * 16
        embv = [idxv[t, pl.ds(r0, 16)] + _OFF_EMB[t] for t in range(6)]
        mv = scal[0, pl.ds(r0, 16)]
        pv = scal[1, pl.ds(r0, 16)]
        bv = [boolv[j, pl.ds(r0, 16)] for j in range(NB)]

        for par, ob, sem in ((0, ob0, sem0), (1, ob1, sem1)):
            c = c2 * 2 + par

            @pl.when(c2 > 0)
            def _wait():
                pltpu.make_async_copy(
                    ob, out_hbm.at[pl.ds(0, _CH)], sem).wait()

            for l in range(_CH):
                lane = par * _CH + l
                m = jnp.full((16,), mv[lane], jnp.float32)
                p = jnp.full((16,), pv[lane], jnp.float32)
                tix = [embv[t][lane] for t in range(6)]
                for k in range(D // 16):
                    sl = pl.ds(k * 16, 16)
                    ob[l, 1, sl] = tv[_OFF_WM, sl] * m + tv[_OFF_BM, sl]
                    ob[l, 2, sl] = tv[_OFF_WP, sl] * p + tv[_OFF_BP, sl]
                    for t in range(6):
                        ob[l, 3 + t, sl] = tv[tix[t], sl]

            for kh in range(2):
                wb = [[tv[_OFF_WB + j, pl.ds((kh * 4 + k) * 16, 16)]
                       for j in range(NB)] for k in range(4)]
                for l in range(_CH):
                    lane = par * _CH + l
                    bs = [bv[j][lane] for j in range(NB)]
                    for k in range(4):
                        sl = pl.ds((kh * 4 + k) * 16, 16)
                        acc = tv[_OFF_BB, sl]
                        for j in range(NB):
                            acc = acc + wb[k][j] * bs[j]
                        ob[l, 9, sl] = acc

            pltpu.async_copy(
                ob, out_hbm.at[pl.ds(base + c * _CH, _CH)], sem)
        return carry

    lax.fori_loop(0, _NPAIR, pair_body, 0)
    pltpu.make_async_copy(ob0, out_hbm.at[pl.ds(0, _CH)], sem0).wait()
    pltpu.make_async_copy(ob1, out_hbm.at[pl.ds(0, _CH)], sem1).wait()


@jax.jit
def _run(T, s1, s2, o1, o2, g1, g2, miss, pc, boolT):
    call = functools.partial(
        pl.kernel,
        out_type=jax.ShapeDtypeStruct((B, NTOK, D), jnp.float32),
        mesh=plsc.VectorSubcoreMesh(core_axis_name="c", subcore_axis_name="s"),
        compiler_params=pltpu.CompilerParams(
            needs_layout_passes=False, use_tc_tiling_on_sc=False),
        scratch_types=[
            pltpu.VMEM((_TROWS, D), jnp.float32),
            pltpu.VMEM((6, _CPW), jnp.int32),
            pltpu.VMEM((2, _CPW), jnp.float32),
            pltpu.VMEM((NB, _CPW), jnp.float32),
            pltpu.VMEM((_CH, NTOK, D), jnp.float32),
            pltpu.VMEM((_CH, NTOK, D), jnp.float32),
            pltpu.SemaphoreType.DMA,
            pltpu.SemaphoreType.DMA,
        ],
    )(_body)
    return call(T, s1, s2, o1, o2, g1, g2, miss, pc, boolT)


def kernel(miss_distance, pc, sat1_type, sat2_type, obj1_type, obj2_type,
           org1, org2, bool_features, W_miss, b_miss, W_pc, b_pc,
           E_sat1, E_sat2, E_obj1, E_obj2, E_org1, E_org2, W_bool, b_bool,
           CLS):
    T = jnp.concatenate([
        CLS.reshape(1, D), W_miss, W_pc,
        E_sat1, E_sat2, E_obj1, E_obj2, E_org1, E_org2,
        b_bool.reshape(1, D), W_bool,
        b_miss.reshape(1, D), b_pc.reshape(1, D),
    ], axis=0)
    return _run(
        T,
        sat1_type.astype(jnp.int32), sat2_type.astype(jnp.int32),
        obj1_type.astype(jnp.int32), obj2_type.astype(jnp.int32),
        org1.astype(jnp.int32), org2.astype(jnp.int32),
        miss_distance.reshape(B), pc.reshape(B),
        bool_features.T.copy())


# speedup vs baseline: 4.2283x; 1.7797x over previous
import functools

import jax
import jax.numpy as jnp
from jax import lax
from jax.experimental import pallas as pl
from jax.experimental.pallas import tpu as pltpu
from jax.experimental.pallas import tpu_sc as plsc

B = 16384
D = 128
NTOK = 10
NB = 10
NK = D // 16

_NC = 2
_NS = 16
_NW = _NC * _NS
_CPW = B // _NW
_CH = 8
_NPAIR = _CPW // (2 * _CH)

_OFF_CLS = 0
_OFF_WM = 1
_OFF_WP = 2
_OFF_EMB = (3, 6, 9, 12, 15, 25)
_OFF_BB = 35
_OFF_WB = 36
_OFF_BM = 46
_OFF_BP = 47
_TROWS = 48


def _sl(k):
    return pl.ds(k * 16, 16)


def _body(T_hbm, s1_hbm, s2_hbm, o1_hbm, o2_hbm, g1_hbm, g2_hbm,
          miss_hbm, pc_hbm, boolT_hbm, out_hbm,
          tv, idxv, scal, boolv, clsb,
          ob1, ob2, ob3, ob4, ob5, ob6, ob7, ob8, ob9, sem0, sem1):
    wid = lax.axis_index("s") * _NC + lax.axis_index("c")
    base = wid * _CPW
    obt = (ob1, ob2, ob3, ob4, ob5, ob6, ob7, ob8, ob9)
    sems = (sem0, sem1)

    pltpu.sync_copy(T_hbm, tv)
    pltpu.sync_copy(s1_hbm.at[pl.ds(base, _CPW)], idxv.at[0])
    pltpu.sync_copy(s2_hbm.at[pl.ds(base, _CPW)], idxv.at[1])
    pltpu.sync_copy(o1_hbm.at[pl.ds(base, _CPW)], idxv.at[2])
    pltpu.sync_copy(o2_hbm.at[pl.ds(base, _CPW)], idxv.at[3])
    pltpu.sync_copy(g1_hbm.at[pl.ds(base, _CPW)], idxv.at[4])
    pltpu.sync_copy(g2_hbm.at[pl.ds(base, _CPW)], idxv.at[5])
    pltpu.sync_copy(miss_hbm.at[pl.ds(base, _CPW)], scal.at[0])
    pltpu.sync_copy(pc_hbm.at[pl.ds(base, _CPW)], scal.at[1])
    pltpu.sync_copy(boolT_hbm.at[:, pl.ds(base, _CPW)], boolv)

    for k in range(NK):
        v = tv[_OFF_CLS, _sl(k)]
        for l in range(_CH):
            clsb[l, _sl(k)] = v

    def pair_body(c2, carry):
        for par in (0, 1):
            c = c2 * 2 + par
            r0 = c * _CH
            gbase = base + r0
            sem = sems[par]

            @pl.when(c2 > 0)
            def _reclaim():
                pltpu.make_async_copy(
                    clsb, out_hbm.at[pl.ds(0, _CH), 0], sem).wait()
                for t in range(1, NTOK):
                    pltpu.make_async_copy(
                        obt[t - 1].at[pl.ds(0, _CH)],
                        out_hbm.at[pl.ds(0, _CH), t], sem).wait()

            wm = [tv[_OFF_WM, _sl(k)] for k in range(NK)]
            bm = [tv[_OFF_BM, _sl(k)] for k in range(NK)]
            wp = [tv[_OFF_WP, _sl(k)] for k in range(NK)]
            bp = [tv[_OFF_BP, _sl(k)] for k in range(NK)]

            def row_a(l, cc):
                row = par * _CH + l
                rs = jnp.full((16,), r0 + l, jnp.int32)
                mg = plsc.load_gather(scal, [jnp.full((16,), 0, jnp.int32), rs])
                pg = plsc.load_gather(scal, [jnp.full((16,), 1, jnp.int32), rs])
                tix = [plsc.load_gather(
                    idxv, [jnp.full((16,), t, jnp.int32), rs])[0] + _OFF_EMB[t]
                    for t in range(6)]
                for k in range(NK):
                    sl = _sl(k)
                    ob1[row, sl] = wm[k] * mg + bm[k]
                    ob2[row, sl] = wp[k] * pg + bp[k]
                    for t in range(6):
                        obt[t + 2][row, sl] = tv[tix[t], sl]
                return cc

            lax.fori_loop(0, _CH, row_a, 0, unroll=2)

            for kh in range(2):
                wb = [[tv[_OFF_WB + j, _sl(kh * 4 + k)] for j in range(NB)]
                      for k in range(4)]
                bb = [tv[_OFF_BB, _sl(kh * 4 + k)] for k in range(4)]

                def row_b(l, cc):
                    row = par * _CH + l
                    rs = jnp.full((16,), r0 + l, jnp.int32)
                    bg = [plsc.load_gather(
                        boolv, [jnp.full((16,), j, jnp.int32), rs])
                        for j in range(NB)]
                    for k in range(4):
                        ps = [wb[k][j] * bg[j] for j in range(NB)]
                        while len(ps) > 1:
                            nxt = [ps[i] + ps[i + 1]
                                   for i in range(0, len(ps) - 1, 2)]
                            if len(ps) % 2:
                                nxt.append(ps[-1])
                            ps = nxt
                        ob9[row, _sl(kh * 4 + k)] = ps[0] + bb[k]
                    return cc

                lax.fori_loop(0, _CH, row_b, 0)

            pltpu.async_copy(clsb, out_hbm.at[pl.ds(gbase, _CH), 0], sem)
            for t in range(1, NTOK):
                pltpu.async_copy(
                    obt[t - 1].at[pl.ds(par * _CH, _CH)],
                    out_hbm.at[pl.ds(gbase, _CH), t], sem)
        return carry

    lax.fori_loop(0, _NPAIR, pair_body, 0)

    for par in (0, 1):
        pltpu.make_async_copy(
            clsb, out_hbm.at[pl.ds(0, _CH), 0], sems[par]).wait()
        for t in range(1, NTOK):
            pltpu.make_async_copy(
                obt[t - 1].at[pl.ds(0, _CH)],
                out_hbm.at[pl.ds(0, _CH), t], sems[par]).wait()


@jax.jit
def _run(T, s1, s2, o1, o2, g1, g2, miss, pc, boolT):
    call = functools.partial(
        pl.kernel,
        out_type=jax.ShapeDtypeStruct((B, NTOK, D), jnp.float32),
        mesh=plsc.VectorSubcoreMesh(core_axis_name="c", subcore_axis_name="s"),
        compiler_params=pltpu.CompilerParams(needs_layout_passes=False),
        scratch_types=(
            [pltpu.VMEM((_TROWS, D), jnp.float32),
             pltpu.VMEM((6, _CPW), jnp.int32),
             pltpu.VMEM((2, _CPW), jnp.float32),
             pltpu.VMEM((NB, _CPW), jnp.float32),
             pltpu.VMEM((_CH, D), jnp.float32)]
            + [pltpu.VMEM((2 * _CH, D), jnp.float32) for _ in range(9)]
            + [pltpu.SemaphoreType.DMA, pltpu.SemaphoreType.DMA]
        ),
    )(_body)
    return call(T, s1, s2, o1, o2, g1, g2, miss, pc, boolT)


def kernel(miss_distance, pc, sat1_type, sat2_type, obj1_type, obj2_type,
           org1, org2, bool_features, W_miss, b_miss, W_pc, b_pc,
           E_sat1, E_sat2, E_obj1, E_obj2, E_org1, E_org2, W_bool, b_bool,
           CLS):
    T = jnp.concatenate([
        CLS.reshape(1, D), W_miss, W_pc,
        E_sat1, E_sat2, E_obj1, E_obj2, E_org1, E_org2,
        b_bool.reshape(1, D), W_bool,
        b_miss.reshape(1, D), b_pc.reshape(1, D),
    ], axis=0)
    return _run(
        T,
        sat1_type.astype(jnp.int32), sat2_type.astype(jnp.int32),
        obj1_type.astype(jnp.int32), obj2_type.astype(jnp.int32),
        org1.astype(jnp.int32), org2.astype(jnp.int32),
        miss_distance.reshape(B), pc.reshape(B),
        bool_features.T.copy())
